# baseline (speedup 1.0000x reference)
import functools

import jax
import jax.numpy as jnp
from jax import lax
from jax.experimental import pallas as pl
from jax.experimental.pallas import tpu as pltpu
from jax.experimental.pallas import tpu_sc as plsc

_NC = 2
_NS = 16
_NW = _NC * _NS
_IDX_CHUNK = 128
_L = 16

_PI = 3.14159265358979
_TWO_PI = 6.283185307179586
_INV_2PI = 0.15915493667125702
_RED_C1 = 6.283203125
_RED_C2 = -1.7642974853515625e-05
_RED_C3 = -1.7484555314695172e-07
_COS_POLY = (0.9999994437335175, -0.49999558241466635, 0.04166103364082131,
             -0.0013862750367048366, 2.4253235371477696e-05,
             -2.219415543283559e-07)


def _poly_cos(x):
    t = x * _INV_2PI
    n = t.astype(jnp.int32).astype(jnp.float32)
    r = x - n * _RED_C1
    r = r - n * _RED_C2
    r = r - n * _RED_C3
    r = r - jnp.where(r > _PI, _TWO_PI, 0.0)
    r = r + jnp.where(r < -_PI, _TWO_PI, 0.0)
    y = r * r
    acc = y * _COS_POLY[-1] + _COS_POLY[-2]
    for c in _COS_POLY[-3::-1]:
        acc = acc * y + c
    return acc


def _sc_assemble(table, idx3, node_msgs, tsb3, prevb3, time_w, time_b,
                 out_dim):
    B = node_msgs.shape[0]
    D = table.shape[1]
    msg = node_msgs.shape[1]
    tdim = time_w.shape[0]
    b_per_w = B // _NW
    n_chunks = idx3.shape[1]

    mesh = plsc.VectorSubcoreMesh(
        core_axis_name="c", subcore_axis_name="s",
        num_cores=_NC, num_subcores=_NS,
    )

    n_rows = 32
    nn = b_per_w // n_rows
    n_tchunks = tdim // _L

    @functools.partial(
        pl.kernel,
        mesh=mesh,
        out_type=jax.ShapeDtypeStruct((B, out_dim), jnp.float32),
        scratch_types=[
            pltpu.VMEM((n_chunks, _IDX_CHUNK), jnp.int32),
            pltpu.VMEM((2, _IDX_CHUNK, D), jnp.float32),
            pltpu.VMEM((2, n_rows, msg), jnp.float32),
            pltpu.VMEM((2, n_rows, tdim), jnp.float32),
            pltpu.VMEM((b_per_w * _L,), jnp.float32),
            pltpu.VMEM((b_per_w * _L,), jnp.float32),
            pltpu.VMEM((tdim,), jnp.float32),
            pltpu.VMEM((tdim,), jnp.float32),
            pltpu.SemaphoreType.DMA,
            pltpu.SemaphoreType.DMA,
            pltpu.SemaphoreType.DMA,
            pltpu.SemaphoreType.DMA,
            pltpu.SemaphoreType.DMA,
        ],
    )
    def k(table_hbm, idx_hbm, node_hbm, tsb_hbm, prevb_hbm, w_hbm, b_hbm,
          out_hbm, idx_v, grows, nbuf, tbuf, tsb_v, prevb_v, w_v, b_v,
          gsem, gosem, nlsem, nssem, tssem):
        wid = lax.axis_index("s") * _NC + lax.axis_index("c")
        base = wid * b_per_w

        def _nload(c):
            return pltpu.make_async_copy(
                node_hbm.at[pl.ds(base + c * n_rows, n_rows)],
                nbuf.at[c % 2], nlsem)

        def _nstore(c):
            return pltpu.make_async_copy(
                nbuf.at[c % 2],
                out_hbm.at[pl.ds(base + c * n_rows, n_rows), pl.ds(0, msg)],
                nssem)

        def _gload(p):
            return pltpu.make_async_copy(
                table_hbm.at[idx_v.at[p]], grows.at[p % 2], gsem)

        def _gstore(p):
            return pltpu.make_async_copy(
                grows.at[p % 2],
                out_hbm.at[pl.ds(base + p * _IDX_CHUNK, _IDX_CHUNK),
                           pl.ds(msg, D)],
                gosem)

        def _tstore(c):
            return pltpu.make_async_copy(
                tbuf.at[c % 2],
                out_hbm.at[pl.ds(base + c * n_rows, n_rows),
                           pl.ds(msg + D, tdim)],
                tssem)

        _nload(0).start()
        pltpu.sync_copy(idx_hbm.at[wid], idx_v)
        _gload(0).start()
        _gload(1).start()
        pltpu.sync_copy(tsb_hbm.at[wid], tsb_v)
        pltpu.sync_copy(prevb_hbm.at[wid], prevb_v)
        pltpu.sync_copy(w_hbm, w_v)
        pltpu.sync_copy(b_hbm, b_v)
        wcs = [w_v[pl.ds(kk * _L, _L)] for kk in range(n_tchunks)]
        bcs = [b_v[pl.ds(kk * _L, _L)] for kk in range(n_tchunks)]

        for c in range(nn):
            _nload(c).wait()
            _nstore(c).start()
            if c + 1 < nn:
                if c >= 1:
                    _nstore(c - 1).wait()
                _nload(c + 1).start()
            if c % 4 == 2:
                p = c // 4
                _gload(p).wait()
                _gstore(p).start()
                if p + 2 < n_chunks:
                    _gstore(p).wait()
                    _gload(p + 2).start()
            if c >= 2:
                _tstore(c - 2).wait()

            def _row(r, carry, c=c):
                row = c * n_rows + r
                sl = pl.ds(row * _L, _L)
                dtr = tsb_v[sl] - prevb_v[sl]
                for kk in range(n_tchunks):
                    x = dtr * wcs[kk] + bcs[kk]
                    tbuf[c % 2, r, pl.ds(kk * _L, _L)] = _poly_cos(x)
                return carry

            lax.fori_loop(0, n_rows, _row, 0)
            _tstore(c).start()

        _nstore(nn - 2).wait()
        _nstore(nn - 1).wait()
        _gstore(n_chunks - 2).wait()
        _gstore(n_chunks - 1).wait()

        _tstore(nn - 2).wait()
        _tstore(nn - 1).wait()

    return k(table, idx3, node_msgs, tsb3, prevb3, time_w, time_b)


def kernel(node_msgs, eids, ts, prev_ts, edge_table, time_w, time_b):
    B, msg = node_msgs.shape
    out_dim = msg + edge_table.shape[1] + time_w.shape[0]
    b_per_w = B // _NW
    eids_i32 = eids.astype(jnp.int32)
    idx3 = eids_i32.reshape(_NW, b_per_w // _IDX_CHUNK, _IDX_CHUNK)
    tsb3 = jnp.broadcast_to(ts[:, None], (B, _L)).reshape(_NW, b_per_w * _L)
    prevb3 = jnp.broadcast_to(prev_ts[:, None],
                              (B, _L)).reshape(_NW, b_per_w * _L)
    full_msgs = _sc_assemble(edge_table, idx3, node_msgs, tsb3, prevb3,
                             time_w, time_b, out_dim)
    return (full_msgs, ts)


# speedup vs baseline: 1.1765x; 1.1765x over previous
import functools

import jax
import jax.numpy as jnp
from jax import lax
from jax.experimental import pallas as pl
from jax.experimental.pallas import tpu as pltpu
from jax.experimental.pallas import tpu_sc as plsc

_NC = 2
_NS = 16
_NW = _NC * _NS
_IDX_CHUNK = 128


def _sc_gather(table, idx):
    B = idx.shape[0]
    D = table.shape[1]
    b_per_w = B // _NW
    n_chunks = b_per_w // _IDX_CHUNK
    idx3 = idx.reshape(_NW, n_chunks, _IDX_CHUNK)

    mesh = plsc.VectorSubcoreMesh(
        core_axis_name="c", subcore_axis_name="s",
        num_cores=_NC, num_subcores=_NS,
    )

    @functools.partial(
        pl.kernel,
        mesh=mesh,
        out_type=jax.ShapeDtypeStruct((B, D), jnp.float32),
        scratch_types=[
            pltpu.VMEM((n_chunks, _IDX_CHUNK), jnp.int32),
            pltpu.VMEM((b_per_w, D), jnp.float32),
            pltpu.SemaphoreType.DMA,
        ],
    )
    def k(table_hbm, idx_hbm, out_hbm, idx_v, rows_v, sem):
        wid = lax.axis_index("s") * _NC + lax.axis_index("c")
        base = wid * b_per_w
        pltpu.sync_copy(idx_hbm.at[wid], idx_v)
        copies = []
        for c in range(n_chunks):
            cp = pltpu.make_async_copy(
                table_hbm.at[idx_v.at[c]],
                rows_v.at[pl.ds(c * _IDX_CHUNK, _IDX_CHUNK)],
                sem,
            )
            cp.start()
            copies.append(cp)
        for cp in copies:
            cp.wait()
        pltpu.sync_copy(rows_v, out_hbm.at[pl.ds(base, b_per_w)])

    return k(table, idx3)


_INV_2PI = 0.15915493667125702
_RED_C1 = 6.283203125
_RED_C2 = -1.7642974853515625e-05
_RED_C3 = -1.7484555314695172e-07
_COS_POLY = (0.9999994437335175, -0.49999558241466635, 0.04166103364082131,
             -0.0013862750367048366, 2.4253235371477696e-05,
             -2.219415543283559e-07)


def _fast_cos(x):
    n = jnp.round(x * _INV_2PI)
    r = x - n * _RED_C1
    r = r - n * _RED_C2
    r = r - n * _RED_C3
    y = r * r
    acc = jnp.float32(_COS_POLY[-1])
    for c in _COS_POLY[-2::-1]:
        acc = acc * y + jnp.float32(c)
    return acc


def _fuse_body(node_ref, edges_ref, ts_ref, prev_ref, w_ref, b_ref, out_ref):
    msg = node_ref.shape[1]
    edg = edges_ref.shape[1]
    out_ref[:, :msg] = node_ref[...]
    out_ref[:, msg:msg + edg] = edges_ref[...]
    dt = ts_ref[...] - prev_ref[...]
    out_ref[:, msg + edg:] = _fast_cos(dt * w_ref[...] + b_ref[...])


def _tc_fuse(node_msgs, edges_vals, ts, prev_ts, time_w, time_b, block_rows):
    B, msg = node_msgs.shape
    edg = edges_vals.shape[1]
    tdim = time_w.shape[0]
    out_dim = msg + edg + tdim
    grid = (B // block_rows,)
    return pl.pallas_call(
        _fuse_body,
        grid=grid,
        in_specs=[
            pl.BlockSpec((block_rows, msg), lambda i: (i, 0)),
            pl.BlockSpec((block_rows, edg), lambda i: (i, 0)),
            pl.BlockSpec((block_rows, 1), lambda i: (i, 0)),
            pl.BlockSpec((block_rows, 1), lambda i: (i, 0)),
            pl.BlockSpec((1, tdim), lambda i: (0, 0)),
            pl.BlockSpec((1, tdim), lambda i: (0, 0)),
        ],
        out_specs=pl.BlockSpec((block_rows, out_dim), lambda i: (i, 0)),
        out_shape=jax.ShapeDtypeStruct((B, out_dim), jnp.float32),
    )(node_msgs, edges_vals, ts.reshape(B, 1), prev_ts.reshape(B, 1),
      time_w.reshape(1, tdim), time_b.reshape(1, tdim))


def kernel(node_msgs, eids, ts, prev_ts, edge_table, time_w, time_b):
    eids_i32 = eids.astype(jnp.int32)
    edges_vals = _sc_gather(edge_table, eids_i32)
    full_msgs = _tc_fuse(node_msgs, edges_vals, ts, prev_ts, time_w, time_b,
                         block_rows=2048)
    return (full_msgs, ts)


# speedup vs baseline: 1.2494x; 1.0620x over previous
import functools

import jax
import jax.numpy as jnp
from jax import lax
from jax.experimental import pallas as pl
from jax.experimental.pallas import tpu as pltpu
from jax.experimental.pallas import tpu_sc as plsc

_NC = 2
_NS = 16
_NW = _NC * _NS
_IDX_CHUNK = 128


def _sc_gather(table, idx):
    B = idx.shape[0]
    D = table.shape[1]
    b_per_w = B // _NW
    n_chunks = b_per_w // _IDX_CHUNK
    idx3 = idx.reshape(_NW, n_chunks, _IDX_CHUNK)

    mesh = plsc.VectorSubcoreMesh(
        core_axis_name="c", subcore_axis_name="s",
        num_cores=_NC, num_subcores=_NS,
    )

    @functools.partial(
        pl.kernel,
        mesh=mesh,
        out_type=jax.ShapeDtypeStruct((B, D), jnp.float32),
        scratch_types=[
            pltpu.VMEM((n_chunks, _IDX_CHUNK), jnp.int32),
            pltpu.VMEM((b_per_w, D), jnp.float32),
            pltpu.SemaphoreType.DMA,
        ],
    )
    def k(table_hbm, idx_hbm, out_hbm, idx_v, rows_v, sem):
        wid = lax.axis_index("s") * _NC + lax.axis_index("c")
        base = wid * b_per_w
        pltpu.sync_copy(idx_hbm.at[wid], idx_v)
        copies = []
        for c in range(n_chunks):
            cp = pltpu.make_async_copy(
                table_hbm.at[idx_v.at[c]],
                rows_v.at[pl.ds(c * _IDX_CHUNK, _IDX_CHUNK)],
                sem,
            )
            cp.start()
            copies.append(cp)
        for cp in copies:
            cp.wait()
        pltpu.sync_copy(rows_v, out_hbm.at[pl.ds(base, b_per_w)])

    return k(table, idx3)


_INV_2PI = 0.15915493667125702
_RED_C1 = 6.283203125
_RED_C2 = -1.7642974853515625e-05
_RED_C3 = -1.7484555314695172e-07
_COS_POLY = (0.9999994437335175, -0.49999558241466635, 0.04166103364082131,
             -0.0013862750367048366, 2.4253235371477696e-05,
             -2.219415543283559e-07)


def _fast_cos(x):
    n = jnp.round(x * _INV_2PI)
    r = x - n * _RED_C1
    r = r - n * _RED_C2
    r = r - n * _RED_C3
    y = r * r
    acc = jnp.float32(_COS_POLY[-1])
    for c in _COS_POLY[-2::-1]:
        acc = acc * y + jnp.float32(c)
    return acc


def _fuse_body(node_ref, edges_ref, dt_ref, w_ref, b_ref, out_ref):
    msg = node_ref.shape[1]
    edg = edges_ref.shape[1]
    out_ref[:, :msg] = node_ref[...]
    out_ref[:, msg:msg + edg] = edges_ref[...]
    out_ref[:, msg + edg:] = _fast_cos(dt_ref[...] * w_ref[...] + b_ref[...])


def _tc_fuse(node_msgs, edges_vals, dt, time_w, time_b, block_rows):
    B, msg = node_msgs.shape
    edg = edges_vals.shape[1]
    tdim = time_w.shape[0]
    out_dim = msg + edg + tdim
    grid = (B // block_rows,)
    return pl.pallas_call(
        _fuse_body,
        grid=grid,
        in_specs=[
            pl.BlockSpec((block_rows, msg), lambda i: (i, 0)),
            pl.BlockSpec((block_rows, edg), lambda i: (i, 0)),
            pl.BlockSpec((block_rows, 1), lambda i: (i, 0)),
            pl.BlockSpec((1, tdim), lambda i: (0, 0)),
            pl.BlockSpec((1, tdim), lambda i: (0, 0)),
        ],
        out_specs=pl.BlockSpec((block_rows, out_dim), lambda i: (i, 0)),
        out_shape=jax.ShapeDtypeStruct((B, out_dim), jnp.float32),
    )(node_msgs, edges_vals, dt.reshape(B, 1),
      time_w.reshape(1, tdim), time_b.reshape(1, tdim))


def kernel(node_msgs, eids, ts, prev_ts, edge_table, time_w, time_b):
    eids_i32 = eids.astype(jnp.int32)
    edges_vals = _sc_gather(edge_table, eids_i32)
    full_msgs = _tc_fuse(node_msgs, edges_vals, ts - prev_ts, time_w, time_b,
                         block_rows=2048)
    return (full_msgs, ts)
